# baseline (speedup 1.0000x reference)
import functools

import jax
import jax.numpy as jnp
from jax import lax
from jax.experimental import pallas as pl
from jax.experimental.pallas import tpu as pltpu
from jax.experimental.pallas import tpu_sc as plsc

DIM = 64
NC = 2
NS = 16
NW = NC * NS
CHUNK = 128


def _sc_gather(x_flat, table128):
    B = x_flat.shape[0]
    b_per_w = B // NW
    n_chunks = b_per_w // CHUNK
    assert n_chunks % 2 == 0
    mesh = plsc.VectorSubcoreMesh(core_axis_name="c", subcore_axis_name="s")

    @functools.partial(
        pl.kernel,
        mesh=mesh,
        out_type=jax.ShapeDtypeStruct((B, DIM), jnp.float32),
        scratch_types=[
            pltpu.VMEM((2, CHUNK), jnp.int32),
            pltpu.VMEM((2, CHUNK, 128), jnp.float32),
            pltpu.VMEM((2, CHUNK, DIM), jnp.float32),
            pltpu.SemaphoreType.DMA((2,)),
            pltpu.SemaphoreType.DMA((2,)),
            pltpu.SemaphoreType.DMA((2,)),
        ],
    )
    def k(table_hbm, idx_hbm, out_hbm, idx_v, rows_v, out_v,
          sem_i, sem_g, sem_w):
        wid = lax.axis_index("s") * NC + lax.axis_index("c")
        base = wid * b_per_w

        def start_idx(g, b):
            pltpu.async_copy(idx_hbm.at[pl.ds(base + g * CHUNK, CHUNK)],
                             idx_v.at[b], sem_i.at[b])

        def wait_idx(b):
            pltpu.make_async_copy(idx_hbm.at[pl.ds(0, CHUNK)],
                                  idx_v.at[b], sem_i.at[b]).wait()

        def start_gather(b):
            pltpu.async_copy(table_hbm.at[idx_v.at[b]], rows_v.at[b],
                             sem_g.at[b])

        def wait_gather(b):
            pltpu.make_async_copy(table_hbm.at[pl.ds(0, CHUNK)],
                                  rows_v.at[b], sem_g.at[b]).wait()

        def start_write(g, b):
            pltpu.async_copy(out_v.at[b],
                             out_hbm.at[pl.ds(base + g * CHUNK, CHUNK)],
                             sem_w.at[b])

        def wait_write(b):
            pltpu.make_async_copy(out_hbm.at[pl.ds(0, CHUNK)],
                                  out_v.at[b], sem_w.at[b]).wait()

        RU = 8

        def compact(b):
            def rowblk(i, c):
                r0 = i * RU
                for k in range(RU):
                    for j in range(DIM // 16):
                        out_v[b, r0 + k, pl.ds(j * 16, 16)] = \
                            rows_v[b, r0 + k, pl.ds(j * 16, 16)]
                return c

            lax.fori_loop(0, CHUNK // RU, rowblk, 0)

        start_idx(0, 0)
        wait_idx(0)
        start_gather(0)
        start_idx(1, 1)

        def step(g, b):
            wait_gather(b)

            @pl.when(g + 2 < n_chunks)
            def _():
                start_idx(g + 2, b)

            @pl.when(g + 1 < n_chunks)
            def _():
                wait_idx(1 - b)
                start_gather(1 - b)

            @pl.when(g >= 2)
            def _():
                wait_write(b)

            compact(b)
            start_write(g, b)

        def pair(p, c):
            step(2 * p, 0)
            step(2 * p + 1, 1)
            return c

        lax.fori_loop(0, n_chunks // 2, pair, 0)
        wait_write(0)
        wait_write(1)

    return k(table128, x_flat)


def kernel(x, embedding):
    b, h = x.shape
    table128 = jnp.pad(embedding, ((0, 0), (0, 128 - DIM)))
    out = _sc_gather(x.reshape(-1), table128)
    return out.reshape(b, h, DIM)


# speedup vs baseline: 1.0014x; 1.0014x over previous
import functools

import jax
import jax.numpy as jnp
from jax import lax
from jax.experimental import pallas as pl
from jax.experimental.pallas import tpu as pltpu
from jax.experimental.pallas import tpu_sc as plsc

DIM = 64
NC = 2
NS = 16
NW = NC * NS
CW = 128
BLK = 8


def _sc_gather(x2, table128, B):
    XR = x2.shape[0]
    rows_per_w = XR // NW
    n_blk = rows_per_w // BLK
    assert n_blk % 2 == 0
    mesh = plsc.VectorSubcoreMesh(core_axis_name="c", subcore_axis_name="s")

    @functools.partial(
        pl.kernel,
        mesh=mesh,
        out_type=jax.ShapeDtypeStruct((B, DIM), jnp.float32),
        scratch_types=[
            pltpu.VMEM((2, BLK, CW), jnp.int32),
            pltpu.VMEM((2, CW, 128), jnp.float32),
            pltpu.VMEM((2, CW, DIM), jnp.float32),
            pltpu.SemaphoreType.DMA((2,)),
            pltpu.SemaphoreType.DMA((2,)),
            pltpu.SemaphoreType.DMA((2,)),
        ],
    )
    def k(table_hbm, x_hbm, out_hbm, idx_v, rows_v, out_v,
          sem_x, sem_g, sem_w):
        wid = lax.axis_index("s") * NC + lax.axis_index("c")
        xrow0 = wid * rows_per_w

        def start_x(o, ob):
            pltpu.async_copy(x_hbm.at[pl.ds(xrow0 + o * BLK, BLK)],
                             idx_v.at[ob], sem_x.at[ob])

        def wait_x(ob):
            pltpu.make_async_copy(x_hbm.at[pl.ds(0, BLK)],
                                  idx_v.at[ob], sem_x.at[ob]).wait()

        def start_gather(ob, kk, b):
            pltpu.async_copy(table_hbm.at[idx_v.at[ob, kk]],
                             rows_v.at[b], sem_g.at[b])

        def wait_gather(b):
            pltpu.make_async_copy(table_hbm.at[pl.ds(0, CW)],
                                  rows_v.at[b], sem_g.at[b]).wait()

        def start_write(g, b):
            pltpu.async_copy(out_v.at[b],
                             out_hbm.at[pl.ds((xrow0 + g) * CW, CW)],
                             sem_w.at[b])

        def wait_write(b):
            pltpu.make_async_copy(out_hbm.at[pl.ds(0, CW)],
                                  out_v.at[b], sem_w.at[b]).wait()

        RU = 8

        def compact_rows(b):
            def rowblk(i, cc):
                r0 = i * RU
                for kk in range(RU):
                    for j in range(DIM // 16):
                        out_v[b, r0 + kk, pl.ds(j * 16, 16)] = \
                            rows_v[b, r0 + kk, pl.ds(j * 16, 16)]
                return cc

            lax.fori_loop(0, CW // RU, rowblk, 0)

        start_x(0, 0)
        wait_x(0)
        start_gather(0, 0, 0)
        start_x(1, 1)

        def block(o, ob):
            for kk in range(BLK):
                g = o * BLK + kk
                b = kk % 2
                wait_gather(b)
                if kk < BLK - 1:
                    start_gather(ob, kk + 1, 1 - b)
                else:
                    @pl.when(o + 1 < n_blk)
                    def _():
                        wait_x(1 - ob)
                        start_gather(1 - ob, 0, 1 - b)

                    @pl.when(o + 2 < n_blk)
                    def _():
                        start_x(o + 2, ob)

                @pl.when(g >= 2)
                def _():
                    wait_write(b)

                compact_rows(b)
                start_write(g, b)

        def bpair(p, cc):
            block(2 * p, 0)
            block(2 * p + 1, 1)
            return cc

        lax.fori_loop(0, n_blk // 2, bpair, 0)
        wait_write(0)
        wait_write(1)

    return k(table128, x2)


def kernel(x, embedding):
    b, h = x.shape
    table128 = jnp.pad(embedding, ((0, 0), (0, 128 - DIM)))
    x2 = x.reshape(-1).reshape(b * h // CW, CW)
    out = _sc_gather(x2, table128, b * h)
    return out.reshape(b, h, DIM)


# speedup vs baseline: 1.0117x; 1.0102x over previous
import functools

import jax
import jax.numpy as jnp
from jax import lax
from jax.experimental import pallas as pl
from jax.experimental.pallas import tpu as pltpu
from jax.experimental.pallas import tpu_sc as plsc

DIM = 64
NC = 2
NS = 16
NW = NC * NS
CW = 128
SUP = 32


def _sc_gather(x, table128):
    R, H = x.shape
    B = R * H
    rows_per_w = R // NW
    n_sup = rows_per_w // SUP
    n_ch = SUP * H // CW
    assert n_sup % 2 == 0 and n_ch % 2 == 0 and (SUP * H) % CW == 0
    mesh = plsc.VectorSubcoreMesh(core_axis_name="c", subcore_axis_name="s")

    @functools.partial(
        pl.kernel,
        mesh=mesh,
        out_type=jax.ShapeDtypeStruct((B, DIM), jnp.float32),
        scratch_types=[
            pltpu.VMEM((2, SUP, H), jnp.int32),
            pltpu.VMEM((2, n_ch, CW), jnp.int32),
            pltpu.VMEM((2, CW, 128), jnp.float32),
            pltpu.VMEM((2, CW, DIM), jnp.float32),
            pltpu.SemaphoreType.DMA((2,)),
            pltpu.SemaphoreType.DMA((2,)),
            pltpu.SemaphoreType.DMA((2,)),
        ],
    )
    def k(table_hbm, x_hbm, out_hbm, raw_v, flat_v, rows_v, out_v,
          sem_x, sem_g, sem_w):
        wid = lax.axis_index("s") * NC + lax.axis_index("c")
        xrow0 = wid * rows_per_w
        obase = xrow0 * H

        li = lax.iota(jnp.int32, 16)
        mlo = li < 8
        mhi = li >= 8

        def start_x(s, rb):
            pltpu.async_copy(x_hbm.at[pl.ds(xrow0 + s * SUP, SUP)],
                             raw_v.at[rb], sem_x.at[rb])

        def wait_x(rb):
            pltpu.make_async_copy(x_hbm.at[pl.ds(0, SUP)],
                                  raw_v.at[rb], sem_x.at[rb]).wait()

        def compact_idx(rb, fb):
            for m in range(SUP * H // 16):
                q = 16 * m
                r, o = q // H, q % H
                cc, lane = q // CW, q % CW
                if o <= H - 16:
                    flat_v[fb, cc, pl.ds(lane, 16)] = \
                        raw_v[rb, r, pl.ds(o, 16)]
                else:
                    a = raw_v[rb, r, pl.ds(H - 16, 16)]
                    bv = raw_v[rb, r + 1, pl.ds(0, 16)]
                    hi = a.at[jnp.minimum(li + 8, 15)].get(
                        mode="promise_in_bounds")
                    lo = bv.at[jnp.maximum(li - 8, 0)].get(
                        mode="promise_in_bounds")
                    flat_v[fb, cc, pl.ds(lane, 16)] = jnp.where(mlo, hi, lo)

        def start_gather(fb, c, b):
            pltpu.async_copy(table_hbm.at[flat_v.at[fb, c]],
                             rows_v.at[b], sem_g.at[b])

        def wait_gather(b):
            pltpu.make_async_copy(table_hbm.at[pl.ds(0, CW)],
                                  rows_v.at[b], sem_g.at[b]).wait()

        def start_write(s, c, b):
            pltpu.async_copy(out_v.at[b],
                             out_hbm.at[pl.ds(obase + (s * n_ch + c) * CW,
                                              CW)],
                             sem_w.at[b])

        def wait_write(b):
            pltpu.make_async_copy(out_hbm.at[pl.ds(0, CW)],
                                  out_v.at[b], sem_w.at[b]).wait()

        RU = 8

        def compact_rows(b):
            def rowblk(i, cc):
                r0 = i * RU
                for kk in range(RU):
                    for j in range(DIM // 16):
                        out_v[b, r0 + kk, pl.ds(j * 16, 16)] = \
                            rows_v[b, r0 + kk, pl.ds(j * 16, 16)]
                return cc

            lax.fori_loop(0, CW // RU, rowblk, 0)

        def sup_body(s, sb):
            @pl.when(s + 1 < n_sup)
            def _():
                wait_x(1 - sb)

            if True:
                @pl.when(s + 1 < n_sup)
                def _():
                    compact_idx(1 - sb, 1 - sb)

                @pl.when(s + 2 < n_sup)
                def _():
                    start_x(s + 2, sb)

            def step(c, b):
                wait_gather(b)

                @pl.when(c + 1 < n_ch)
                def _():
                    start_gather(sb, c + 1, 1 - b)

                @pl.when((c + 1 >= n_ch) & (s + 1 < n_sup))
                def _():
                    start_gather(1 - sb, 0, 1 - b)

                @pl.when((s > 0) | (c >= 2))
                def _():
                    wait_write(b)

                compact_rows(b)
                start_write(s, c, b)

            def chpair(p, cc):
                step(2 * p, 0)
                step(2 * p + 1, 1)
                return cc

            lax.fori_loop(0, n_ch // 2, chpair, 0)

        start_x(0, 0)
        wait_x(0)
        start_x(1, 1)
        compact_idx(0, 0)
        start_gather(0, 0, 0)

        def sup_pair(sp, cc):
            sup_body(2 * sp, 0)
            sup_body(2 * sp + 1, 1)
            return cc

        lax.fori_loop(0, n_sup // 2, sup_pair, 0)
        wait_write(0)
        wait_write(1)

    return k(table128, x)


def kernel(x, embedding):
    b, h = x.shape
    table128 = jnp.pad(embedding, ((0, 0), (0, 128 - DIM)))
    out = _sc_gather(x, table128)
    return out.reshape(b, h, DIM)
